# Initial kernel scaffold; baseline (speedup 1.0000x reference)
import jax
import jax.numpy as jnp
from jax.experimental import pallas as pl


def kernel(x, edge_index, batch, W1, b1, g1, be1, W2, b2, g2, be2, gc, bcn, Wc1, bc1, Wc2, bc2):
    raise NotImplementedError("write your pallas kernel here")


# speedup vs baseline: 2.7225x; 2.7225x over previous
import functools

import jax
import jax.numpy as jnp
from jax import lax
from jax.experimental import pallas as pl
from jax.experimental.pallas import tpu as pltpu
from jax.experimental.pallas import tpu_sc as plsc

_L = 128
_NW = 32
_NTILES = 16


def _make_sc_agg(N, D, CH, NPAD):
    FULL = ((N + _NTILES - 1) // _NTILES + 7) // 8 * 8
    LAST = N - (_NTILES - 1) * FULL
    mesh = plsc.VectorSubcoreMesh(core_axis_name="c", subcore_axis_name="s")

    @functools.partial(
        pl.kernel, mesh=mesh,
        out_type=jax.ShapeDtypeStruct((2, N, D), jnp.float32),
        scratch_types=[
            pltpu.VMEM((CH, _L), jnp.int32),
            pltpu.VMEM((CH, _L), jnp.int32),
            pltpu.VMEM((_L, D), jnp.float32),
            pltpu.VMEM_SHARED((NPAD, D), jnp.float32),
            pltpu.SemaphoreType.DMA,
        ],
    )
    def sc_agg(h_hbm, src_hbm, dst_hbm, out_hbm, src_v, dst_v, rows_v, acc_sh,
               sem):
        c = lax.axis_index("c")
        s = lax.axis_index("s")
        w = c * _NTILES + s
        pltpu.sync_copy(src_hbm.at[pl.ds(w * CH, CH)], src_v)
        pltpu.sync_copy(dst_hbm.at[pl.ds(w * CH, CH)], dst_v)
        r0 = pl.multiple_of(s * FULL, 8)

        @pl.when(s < _NTILES - 1)
        def _():
            pltpu.sync_copy(h_hbm.at[pl.ds(r0, FULL)],
                            acc_sh.at[pl.ds(r0, FULL)])

        @pl.when(s == _NTILES - 1)
        def _():
            pltpu.sync_copy(h_hbm.at[pl.ds(r0, LAST)],
                            acc_sh.at[pl.ds(r0, LAST)])

        plsc.subcore_barrier()

        def body(j, carry):
            pltpu.async_copy(h_hbm.at[src_v.at[j]], rows_v, sem).wait()
            pltpu.sync_copy(rows_v, acc_sh.at[dst_v.at[j]], add=True)
            return carry

        lax.fori_loop(0, CH, body, 0)
        plsc.subcore_barrier()

        @pl.when(s < _NTILES - 1)
        def _():
            pltpu.sync_copy(acc_sh.at[pl.ds(r0, FULL)],
                            out_hbm.at[c, pl.ds(r0, FULL)])

        @pl.when(s == _NTILES - 1)
        def _():
            pltpu.sync_copy(acc_sh.at[pl.ds(r0, LAST)],
                            out_hbm.at[c, pl.ds(r0, LAST)])

    return sc_agg


def _mlp_body(G, N, h_ref, a0_ref, a1_ref, b_ref, w1_ref, b1_ref, g1_ref,
              be1_ref, w2_ref, b2_ref, g2_ref, be2_ref, hout_ref, pool_ref):
    y = a0_ref[...] + a1_ref[...] - h_ref[...]
    h1 = jnp.maximum(
        jnp.dot(y, w1_ref[...], preferred_element_type=jnp.float32)
        + b1_ref[...], 0.0)
    m1 = jnp.mean(h1, axis=0, keepdims=True)
    v1 = jnp.mean((h1 - m1) ** 2, axis=0, keepdims=True)
    h1 = (h1 - m1) * lax.rsqrt(v1 + 1e-5) * g1_ref[...] + be1_ref[...]
    h2 = jnp.maximum(
        jnp.dot(h1, w2_ref[...], preferred_element_type=jnp.float32)
        + b2_ref[...], 0.0)
    m2 = jnp.mean(h2, axis=0, keepdims=True)
    v2 = jnp.mean((h2 - m2) ** 2, axis=0, keepdims=True)
    h2 = (h2 - m2) * lax.rsqrt(v2 + 1e-5) * g2_ref[...] + be2_ref[...]
    hout_ref[...] = h2
    gids = lax.broadcasted_iota(jnp.int32, (G, N), 0)
    onehot = (gids == b_ref[...]).astype(jnp.float32)
    pool_ref[...] = jnp.dot(onehot, h2, preferred_element_type=jnp.float32)


def _cls_body(f_ref, gc_ref, bcn_ref, w1_ref, b1_ref, w2_ref, b2_ref, o_ref):
    f = f_ref[...]
    m = jnp.mean(f, axis=0, keepdims=True)
    v = jnp.mean((f - m) ** 2, axis=0, keepdims=True)
    f = (f - m) * lax.rsqrt(v + 1e-5) * gc_ref[...] + bcn_ref[...]
    f = jnp.maximum(
        jnp.dot(f, w1_ref[...], preferred_element_type=jnp.float32)
        + b1_ref[...], 0.0)
    z = jnp.dot(f, w2_ref[...], preferred_element_type=jnp.float32) + b2_ref[...]
    zm = jnp.max(z, axis=-1, keepdims=True)
    o_ref[...] = (z - zm) - jnp.log(
        jnp.sum(jnp.exp(z - zm), axis=-1, keepdims=True))


def kernel(x, edge_index, batch, W1, b1, g1, be1, W2, b2, g2, be2, gc, bcn,
           Wc1, bc1, Wc2, bc2):
    N, D = x.shape
    E = edge_index.shape[1]
    BLOCKS = W1.shape[0]
    G = 64
    C = Wc2.shape[1]

    CH = -(-E // (_NW * _L))
    if CH % 2:
        CH += 1
    Epad = _NW * CH * _L
    NPAD = N + 16
    src = edge_index[0]
    dst = edge_index[1]
    pad = Epad - E
    srcp = jnp.concatenate(
        [src, jnp.zeros((pad,), jnp.int32)]).reshape(_NW * CH, _L)
    dstp = jnp.concatenate(
        [dst, jnp.full((pad,), N, jnp.int32)]).reshape(_NW * CH, _L)
    batch_row = batch.reshape(1, N)

    sc_agg = _make_sc_agg(N, D, CH, NPAD)

    mlp = pl.pallas_call(
        functools.partial(_mlp_body, G, N),
        out_shape=[
            jax.ShapeDtypeStruct((N, D), jnp.float32),
            jax.ShapeDtypeStruct((G, D), jnp.float32),
        ],
    )

    cls = pl.pallas_call(
        _cls_body,
        out_shape=jax.ShapeDtypeStruct((G, C), jnp.float32),
    )

    h = x
    pooled = []
    for i in range(BLOCKS):
        acc = sc_agg(h, srcp, dstp)
        h, pool_i = mlp(h, acc[0], acc[1], batch_row,
                        W1[i], b1[i].reshape(1, D), g1[i].reshape(1, D),
                        be1[i].reshape(1, D),
                        W2[i], b2[i].reshape(1, D), g2[i].reshape(1, D),
                        be2[i].reshape(1, D))
        pooled.append(pool_i)

    f = jnp.concatenate(pooled, axis=1)
    return cls(f, gc.reshape(1, -1), bcn.reshape(1, -1), Wc1,
               bc1.reshape(1, -1), Wc2, bc2.reshape(1, -1))
